# Initial kernel scaffold; baseline (speedup 1.0000x reference)
import jax
import jax.numpy as jnp
from jax.experimental import pallas as pl


def kernel(x, edge_index, edge_attr, batch, W_node, b_node, conv_W, conv_b, We1, be1, We2, be2, W1, b1, W2, b2):
    raise NotImplementedError("write your pallas kernel here")


# speedup vs baseline: 5.2271x; 5.2271x over previous
import functools

import jax
import jax.numpy as jnp
from jax import lax
from jax.experimental import pallas as pl
from jax.experimental.pallas import tpu as pltpu
from jax.experimental.pallas import tpu_sc as plsc

N = 10000
E = 320000
D = 128
DE = 16
H = 128
G = 64

NC = 2
NS = 16
NW = NC * NS
EPT = E // NW
BB = 80
NBLK = EPT // BB
NP = 10240
RPT = NP // NS
NCH = RPT // 128

_mesh = plsc.VectorSubcoreMesh(core_axis_name="c", subcore_axis_name="s")


def _fill_row_ids(buf, base):
    for k in range(8):
        buf[pl.ds(k * 16, 16)] = (
            lax.broadcast_in_dim(base + k * 16, (16,), ())
            + lax.iota(jnp.int32, 16)
        )


def _sc_degree(dst):
    @functools.partial(
        pl.kernel,
        out_type=jax.ShapeDtypeStruct((NC, NP, H), jnp.float32),
        mesh=_mesh,
        scratch_types=[
            pltpu.VMEM((BB,), jnp.int32),
            pltpu.VMEM((BB, H), jnp.float32),
            pltpu.VMEM((128, H), jnp.float32),
        ]
        + [pltpu.VMEM((128,), jnp.int32) for _ in range(NCH)]
        + [pltpu.VMEM_SHARED((NP, H), jnp.float32)],
    )
    def k(dst_hbm, out_hbm, idx_v, ones_v, zbuf, *rest):
        own = rest[:NCH]
        acc = rest[NCH]
        c = lax.axis_index("c")
        s = lax.axis_index("s")

        @pl.loop(0, BB)
        def _(r):
            for ch in range(H // 16):
                ones_v[r, pl.ds(ch * 16, 16)] = jnp.ones((16,), jnp.float32)

        @pl.loop(0, 128)
        def _(r):
            for ch in range(H // 16):
                zbuf[r, pl.ds(ch * 16, 16)] = jnp.zeros((16,), jnp.float32)

        for t in range(NCH):
            _fill_row_ids(own[t], s * RPT + t * 128)
            pltpu.sync_copy(zbuf, acc.at[own[t]])
        plsc.subcore_barrier()

        tile_base = (c * NS + s) * EPT

        @pl.loop(0, NBLK)
        def _(j):
            pltpu.sync_copy(dst_hbm.at[pl.ds(tile_base + j * BB, BB)], idx_v)
            pltpu.sync_copy(ones_v, acc.at[idx_v], add=True)

        plsc.subcore_barrier()
        for t in range(NCH):
            pltpu.sync_copy(acc.at[own[t]], zbuf)
            pltpu.sync_copy(zbuf, out_hbm.at[c, pl.ds(s * RPT + t * 128, 128)])

    return k(dst)


def _sc_gather_mul_scatter(hwp, ew, src, dst):
    @functools.partial(
        pl.kernel,
        out_type=jax.ShapeDtypeStruct((NC, NP, H), jnp.float32),
        mesh=_mesh,
        scratch_types=[
            pltpu.VMEM((BB,), jnp.int32),
            pltpu.VMEM((BB,), jnp.int32),
            pltpu.VMEM((BB, H), jnp.float32),
            pltpu.VMEM((BB, H), jnp.float32),
            pltpu.VMEM((128, H), jnp.float32),
        ]
        + [pltpu.VMEM((128,), jnp.int32) for _ in range(NCH)]
        + [pltpu.VMEM_SHARED((NP, H), jnp.float32)],
    )
    def k(hw_hbm, e_hbm, src_hbm, dst_hbm, out_hbm,
          idx_s, idx_d, gbuf, ebuf, zbuf, *rest):
        own = rest[:NCH]
        acc = rest[NCH]
        c = lax.axis_index("c")
        s = lax.axis_index("s")

        @pl.loop(0, 128)
        def _(r):
            for ch in range(H // 16):
                zbuf[r, pl.ds(ch * 16, 16)] = jnp.zeros((16,), jnp.float32)

        for t in range(NCH):
            _fill_row_ids(own[t], s * RPT + t * 128)
            pltpu.sync_copy(zbuf, acc.at[own[t]])
        plsc.subcore_barrier()

        tile_base = (c * NS + s) * EPT

        @pl.loop(0, NBLK)
        def _(j):
            base = tile_base + j * BB
            pltpu.sync_copy(src_hbm.at[pl.ds(base, BB)], idx_s)
            pltpu.sync_copy(dst_hbm.at[pl.ds(base, BB)], idx_d)
            pltpu.sync_copy(hw_hbm.at[idx_s], gbuf)
            pltpu.sync_copy(e_hbm.at[pl.ds(base, BB)], ebuf)

            @pl.loop(0, BB)
            def _(r):
                for ch in range(H // 16):
                    sl = pl.ds(ch * 16, 16)
                    gbuf[r, sl] = gbuf[r, sl] * ebuf[r, sl]

            pltpu.sync_copy(gbuf, acc.at[idx_d], add=True)

        plsc.subcore_barrier()
        for t in range(NCH):
            pltpu.sync_copy(acc.at[own[t]], zbuf)
            pltpu.sync_copy(zbuf, out_hbm.at[c, pl.ds(s * RPT + t * 128, 128)])

    return k(hwp, ew, src, dst)


BE = 4000
BN = 2000


def _tc_edge_mlp(edge_attr, We1, be1, We2, be2):
    def body(ea_ref, w1_ref, b1_ref, w2_ref, b2_ref, o_ref):
        t = jnp.maximum(
            jnp.dot(ea_ref[...], w1_ref[...],
                    preferred_element_type=jnp.float32) + b1_ref[...], 0.0)
        o_ref[...] = jnp.dot(t, w2_ref[...],
                             preferred_element_type=jnp.float32) + b2_ref[...]

    return pl.pallas_call(
        body,
        grid=(E // BE,),
        in_specs=[
            pl.BlockSpec((BE, DE), lambda i: (i, 0)),
            pl.BlockSpec((DE, H), lambda i: (0, 0)),
            pl.BlockSpec((H,), lambda i: (0,)),
            pl.BlockSpec((H, H), lambda i: (0, 0)),
            pl.BlockSpec((H,), lambda i: (0,)),
        ],
        out_specs=pl.BlockSpec((BE, H), lambda i: (i, 0)),
        out_shape=jax.ShapeDtypeStruct((E, H), jnp.float32),
    )(edge_attr, We1, be1, We2, be2)


def _tc_dinv(degp):
    def body(d_ref, o_ref):
        i = pl.program_id(0)
        deg = d_ref[0][:, 0:1] + d_ref[1][:, 0:1] + 1.0
        o_ref[...] = lax.rsqrt(deg)

    return pl.pallas_call(
        body,
        grid=(NP // BN,),
        in_specs=[pl.BlockSpec((NC, BN, H), lambda i: (0, i, 0))],
        out_specs=pl.BlockSpec((BN, 1), lambda i: (i, 0)),
        out_shape=jax.ShapeDtypeStruct((NP, 1), jnp.float32),
    )(degp)


def _tc_embed(x, W_node, b_node, W0, dinv2):
    def body(x_ref, wn_ref, bn_ref, w0_ref, di_ref, o_ref):
        h = jnp.dot(x_ref[...], wn_ref[...],
                    preferred_element_type=jnp.float32) + bn_ref[...]
        o_ref[...] = jnp.dot(h, w0_ref[...],
                             preferred_element_type=jnp.float32) * di_ref[...]

    return pl.pallas_call(
        body,
        grid=(N // BN,),
        in_specs=[
            pl.BlockSpec((BN, D), lambda i: (i, 0)),
            pl.BlockSpec((D, H), lambda i: (0, 0)),
            pl.BlockSpec((H,), lambda i: (0,)),
            pl.BlockSpec((H, H), lambda i: (0, 0)),
            pl.BlockSpec((BN, 1), lambda i: (i, 0)),
        ],
        out_specs=pl.BlockSpec((BN, H), lambda i: (i, 0)),
        out_shape=jax.ShapeDtypeStruct((N, H), jnp.float32),
    )(x, W_node, b_node, W0, dinv2)


def _tc_layer_update(p, hwp, dinv2, b_prev, W_next):
    def body(p_ref, hw_ref, di_ref, b_ref, w_ref, o_ref):
        di = di_ref[...]
        h = jnp.maximum(di * (p_ref[0] + p_ref[1] + hw_ref[...]) + b_ref[...],
                        0.0)
        o_ref[...] = jnp.dot(h, w_ref[...],
                             preferred_element_type=jnp.float32) * di

    return pl.pallas_call(
        body,
        grid=(N // BN,),
        in_specs=[
            pl.BlockSpec((NC, BN, H), lambda i: (0, i, 0)),
            pl.BlockSpec((BN, H), lambda i: (i, 0)),
            pl.BlockSpec((BN, 1), lambda i: (i, 0)),
            pl.BlockSpec((H,), lambda i: (0,)),
            pl.BlockSpec((H, H), lambda i: (0, 0)),
        ],
        out_specs=pl.BlockSpec((BN, H), lambda i: (i, 0)),
        out_shape=jax.ShapeDtypeStruct((N, H), jnp.float32),
    )(p, hwp, dinv2, b_prev, W_next)


def _tc_pool_head(p, hwp, dinv2, b_prev, batch2, W1, b1, W2t, b2):
    nsteps = N // BN

    def body(p_ref, hw_ref, di_ref, b_ref, bat_ref, w1_ref, b1_ref,
             w2_ref, b2_ref, o_ref, sums, counts):
        i = pl.program_id(0)

        @pl.when(i == 0)
        def _():
            sums[...] = jnp.zeros_like(sums)
            counts[...] = jnp.zeros_like(counts)

        di = di_ref[...]
        h = jnp.maximum(di * (p_ref[0] + p_ref[1] + hw_ref[...]) + b_ref[...],
                        0.0)
        gids = jax.lax.broadcasted_iota(jnp.int32, (1, G), 1)
        oh = (bat_ref[...] == gids).astype(jnp.float32)
        sums[...] += lax.dot_general(oh, h, (((0,), (0,)), ((), ())),
                                     preferred_element_type=jnp.float32)
        counts[...] += jnp.sum(oh, axis=0)[:, None]

        @pl.when(i == nsteps - 1)
        def _():
            g = sums[...] / jnp.maximum(counts[...], 1.0)
            z = jnp.maximum(
                jnp.dot(g, w1_ref[...],
                        preferred_element_type=jnp.float32) + b1_ref[...], 0.0)
            o_ref[...] = (jnp.sum(z * w2_ref[...], axis=1, keepdims=True)
                          + b2_ref[...])

    return pl.pallas_call(
        body,
        grid=(nsteps,),
        in_specs=[
            pl.BlockSpec((NC, BN, H), lambda i: (0, i, 0)),
            pl.BlockSpec((BN, H), lambda i: (i, 0)),
            pl.BlockSpec((BN, 1), lambda i: (i, 0)),
            pl.BlockSpec((H,), lambda i: (0,)),
            pl.BlockSpec((BN, 1), lambda i: (i, 0)),
            pl.BlockSpec((H, H), lambda i: (0, 0)),
            pl.BlockSpec((H,), lambda i: (0,)),
            pl.BlockSpec((1, H), lambda i: (0, 0)),
            pl.BlockSpec((1,), lambda i: (0,)),
        ],
        out_specs=pl.BlockSpec((G, 1), lambda i: (0, 0)),
        out_shape=jax.ShapeDtypeStruct((G, 1), jnp.float32),
        scratch_shapes=[
            pltpu.VMEM((G, H), jnp.float32),
            pltpu.VMEM((G, 1), jnp.float32),
        ],
    )(p, hwp, dinv2, b_prev, batch2, W1, b1, W2t, b2)


def kernel(x, edge_index, edge_attr, batch, W_node, b_node, conv_W, conv_b,
           We1, be1, We2, be2, W1, b1, W2, b2):
    src = edge_index[0]
    dst = edge_index[1]

    degp = _sc_degree(dst)
    ew = _tc_edge_mlp(edge_attr, We1, be1, We2, be2)
    dinv2 = _tc_dinv(degp)

    hwp = _tc_embed(x, W_node, b_node, conv_W[0], dinv2)
    for l in range(2):
        p = _sc_gather_mul_scatter(hwp, ew, src, dst)
        hwp = _tc_layer_update(p, hwp, dinv2, conv_b[l], conv_W[l + 1])
    p = _sc_gather_mul_scatter(hwp, ew, src, dst)

    return _tc_pool_head(p, hwp, dinv2, conv_b[2], batch[:, None],
                         W1, b1, W2.reshape(1, H), b2)


# speedup vs baseline: 7.6472x; 1.4630x over previous
import functools

import jax
import jax.numpy as jnp
from jax import lax
from jax.experimental import pallas as pl
from jax.experimental.pallas import tpu as pltpu
from jax.experimental.pallas import tpu_sc as plsc

N = 10000
E = 320000
D = 128
DE = 16
H = 128
G = 64

NC = 2
NS = 16
NW = NC * NS
EPT = E // NW
BB = 80
NBLK = EPT // BB
NP = 10240
RPT = NP // NS
NCH = RPT // BB

_mesh = plsc.VectorSubcoreMesh(core_axis_name="c", subcore_axis_name="s")


def _fill_row_ids(buf, base):
    for k in range(BB // 16):
        buf[pl.ds(k * 16, 16)] = (
            lax.broadcast_in_dim(base + k * 16, (16,), ())
            + lax.iota(jnp.int32, 16)
        )


def _sc_degree(dst):
    @functools.partial(
        pl.kernel,
        out_type=jax.ShapeDtypeStruct((NC, NP, H), jnp.float32),
        mesh=_mesh,
        scratch_types=[
            pltpu.VMEM((BB,), jnp.int32),
            pltpu.VMEM((BB,), jnp.int32),
            pltpu.VMEM((BB, H), jnp.float32),
            pltpu.VMEM((BB, H), jnp.float32),
        ]
        + [pltpu.VMEM((BB,), jnp.int32) for _ in range(NCH)]
        + [pltpu.VMEM_SHARED((NP, H), jnp.float32)]
        + [pltpu.SemaphoreType.DMA for _ in range(2)],
    )
    def k(dst_hbm, out_hbm, idx0, idx1, ones_v, zbuf, *rest):
        own = rest[:NCH]
        acc = rest[NCH]
        sem = rest[NCH + 1:NCH + 3]
        idx = [idx0, idx1]
        c = lax.axis_index("c")
        s = lax.axis_index("s")

        @pl.loop(0, BB)
        def _(r):
            for ch in range(H // 16):
                ones_v[r, pl.ds(ch * 16, 16)] = jnp.ones((16,), jnp.float32)

        @pl.loop(0, BB)
        def _(r):
            for ch in range(H // 16):
                zbuf[r, pl.ds(ch * 16, 16)] = jnp.zeros((16,), jnp.float32)

        for t in range(NCH):
            _fill_row_ids(own[t], s * RPT + t * BB)
            pltpu.sync_copy(zbuf, acc.at[own[t]])
        plsc.subcore_barrier()

        tile_base = (c * NS + s) * EPT

        def start_idx(blk, b):
            pltpu.async_copy(
                dst_hbm.at[pl.ds(tile_base + blk * BB, BB)], idx[b], sem[b])

        def wait_idx(blk, b):
            pltpu.make_async_copy(
                dst_hbm.at[pl.ds(tile_base + blk * BB, BB)], idx[b],
                sem[b]).wait()

        start_idx(0, 0)

        @pl.loop(0, NBLK - 1, step=2)
        def _(j):
            for b in range(2):
                start_idx(j + b + 1, 1 - b)
                wait_idx(j + b, b)
                pltpu.sync_copy(ones_v, acc.at[idx[b]], add=True)

        wait_idx(NBLK - 1, 0)
        pltpu.sync_copy(ones_v, acc.at[idx0], add=True)

        plsc.subcore_barrier()
        for t in range(NCH):
            pltpu.sync_copy(acc.at[own[t]], zbuf)
            pltpu.sync_copy(zbuf, out_hbm.at[c, pl.ds(s * RPT + t * BB, BB)])

    return k(dst)


def _sc_gather_mul_scatter(hwp, ew, src, dst):
    @functools.partial(
        pl.kernel,
        out_type=jax.ShapeDtypeStruct((NC, NP, H), jnp.float32),
        mesh=_mesh,
        scratch_types=[
            pltpu.VMEM((BB,), jnp.int32),
            pltpu.VMEM((BB,), jnp.int32),
            pltpu.VMEM((BB,), jnp.int32),
            pltpu.VMEM((BB,), jnp.int32),
            pltpu.VMEM((BB, H), jnp.float32),
            pltpu.VMEM((BB, H), jnp.float32),
            pltpu.VMEM((BB, H), jnp.float32),
            pltpu.VMEM((BB, H), jnp.float32),
        ]
        + [pltpu.VMEM((BB,), jnp.int32) for _ in range(NCH)]
        + [pltpu.VMEM_SHARED((NP, H), jnp.float32)]
        + [pltpu.SemaphoreType.DMA for _ in range(8)],
    )
    def k(hw_hbm, e_hbm, src_hbm, dst_hbm, out_hbm,
          is0, is1, id0, id1, gb0, gb1, eb0, eb1, *rest):
        own = rest[:NCH]
        acc = rest[NCH]
        sems = rest[NCH + 1:NCH + 9]
        idx_s, idx_d = [is0, is1], [id0, id1]
        gbuf, ebuf = [gb0, gb1], [eb0, eb1]
        s_is, s_id, s_e, s_g = sems[0:2], sems[2:4], sems[4:6], sems[6:8]
        c = lax.axis_index("c")
        s = lax.axis_index("s")

        @pl.loop(0, BB)
        def _(r):
            for ch in range(H // 16):
                gb0[r, pl.ds(ch * 16, 16)] = jnp.zeros((16,), jnp.float32)

        for t in range(NCH):
            _fill_row_ids(own[t], s * RPT + t * BB)
            pltpu.sync_copy(gb0, acc.at[own[t]])
        plsc.subcore_barrier()

        tile_base = (c * NS + s) * EPT

        def start_loads(blk, b):
            base = tile_base + blk * BB
            pltpu.async_copy(src_hbm.at[pl.ds(base, BB)], idx_s[b], s_is[b])
            pltpu.async_copy(dst_hbm.at[pl.ds(base, BB)], idx_d[b], s_id[b])
            pltpu.async_copy(e_hbm.at[pl.ds(base, BB)], ebuf[b], s_e[b])

        def wait_1d(hbm, blk, dstb, sem):
            pltpu.make_async_copy(
                hbm.at[pl.ds(tile_base + blk * BB, BB)], dstb, sem).wait()

        def start_gather(b):
            pltpu.async_copy(hw_hbm.at[idx_s[b]], gbuf[b], s_g[b])

        def finish_block(blk, b):
            wait_1d(e_hbm, blk, ebuf[b], s_e[b])
            pltpu.make_async_copy(hw_hbm.at[idx_s[b]], gbuf[b], s_g[b]).wait()

            @pl.loop(0, BB)
            def _(r):
                for ch in range(H // 16):
                    sl = pl.ds(ch * 16, 16)
                    gbuf[b][r, sl] = gbuf[b][r, sl] * ebuf[b][r, sl]

            wait_1d(dst_hbm, blk, idx_d[b], s_id[b])
            pltpu.sync_copy(gbuf[b], acc.at[idx_d[b]], add=True)

        start_loads(0, 0)
        wait_1d(src_hbm, 0, idx_s[0], s_is[0])
        start_gather(0)

        @pl.loop(0, NBLK - 1, step=2)
        def _(j):
            for b in range(2):
                start_loads(j + b + 1, 1 - b)
                finish_block(j + b, b)
                wait_1d(src_hbm, j + b + 1, idx_s[1 - b], s_is[1 - b])
                start_gather(1 - b)

        finish_block(NBLK - 1, 0)

        plsc.subcore_barrier()
        for t in range(NCH):
            pltpu.sync_copy(acc.at[own[t]], gb0)
            pltpu.sync_copy(gb0, out_hbm.at[c, pl.ds(s * RPT + t * BB, BB)])

    return k(hwp, ew, src, dst)


BE = 4000
BN = 2000


def _tc_edge_mlp(edge_attr, We1, be1, We2, be2):
    def body(ea_ref, w1_ref, b1_ref, w2_ref, b2_ref, o_ref):
        t = jnp.maximum(
            jnp.dot(ea_ref[...], w1_ref[...],
                    preferred_element_type=jnp.float32) + b1_ref[...], 0.0)
        o_ref[...] = jnp.dot(t, w2_ref[...],
                             preferred_element_type=jnp.float32) + b2_ref[...]

    return pl.pallas_call(
        body,
        grid=(E // BE,),
        in_specs=[
            pl.BlockSpec((BE, DE), lambda i: (i, 0)),
            pl.BlockSpec((DE, H), lambda i: (0, 0)),
            pl.BlockSpec((H,), lambda i: (0,)),
            pl.BlockSpec((H, H), lambda i: (0, 0)),
            pl.BlockSpec((H,), lambda i: (0,)),
        ],
        out_specs=pl.BlockSpec((BE, H), lambda i: (i, 0)),
        out_shape=jax.ShapeDtypeStruct((E, H), jnp.float32),
    )(edge_attr, We1, be1, We2, be2)


def _tc_dinv(degp):
    def body(d_ref, o_ref):
        i = pl.program_id(0)
        deg = d_ref[0][:, 0:1] + d_ref[1][:, 0:1] + 1.0
        o_ref[...] = lax.rsqrt(deg)

    return pl.pallas_call(
        body,
        grid=(NP // BN,),
        in_specs=[pl.BlockSpec((NC, BN, H), lambda i: (0, i, 0))],
        out_specs=pl.BlockSpec((BN, 1), lambda i: (i, 0)),
        out_shape=jax.ShapeDtypeStruct((NP, 1), jnp.float32),
    )(degp)


def _tc_embed(x, W_node, b_node, W0, dinv2):
    def body(x_ref, wn_ref, bn_ref, w0_ref, di_ref, o_ref):
        h = jnp.dot(x_ref[...], wn_ref[...],
                    preferred_element_type=jnp.float32) + bn_ref[...]
        o_ref[...] = jnp.dot(h, w0_ref[...],
                             preferred_element_type=jnp.float32) * di_ref[...]

    return pl.pallas_call(
        body,
        grid=(N // BN,),
        in_specs=[
            pl.BlockSpec((BN, D), lambda i: (i, 0)),
            pl.BlockSpec((D, H), lambda i: (0, 0)),
            pl.BlockSpec((H,), lambda i: (0,)),
            pl.BlockSpec((H, H), lambda i: (0, 0)),
            pl.BlockSpec((BN, 1), lambda i: (i, 0)),
        ],
        out_specs=pl.BlockSpec((BN, H), lambda i: (i, 0)),
        out_shape=jax.ShapeDtypeStruct((N, H), jnp.float32),
    )(x, W_node, b_node, W0, dinv2)


def _tc_layer_update(p, hwp, dinv2, b_prev, W_next):
    def body(p_ref, hw_ref, di_ref, b_ref, w_ref, o_ref):
        di = di_ref[...]
        h = jnp.maximum(di * (p_ref[0] + p_ref[1] + hw_ref[...]) + b_ref[...],
                        0.0)
        o_ref[...] = jnp.dot(h, w_ref[...],
                             preferred_element_type=jnp.float32) * di

    return pl.pallas_call(
        body,
        grid=(N // BN,),
        in_specs=[
            pl.BlockSpec((NC, BN, H), lambda i: (0, i, 0)),
            pl.BlockSpec((BN, H), lambda i: (i, 0)),
            pl.BlockSpec((BN, 1), lambda i: (i, 0)),
            pl.BlockSpec((H,), lambda i: (0,)),
            pl.BlockSpec((H, H), lambda i: (0, 0)),
        ],
        out_specs=pl.BlockSpec((BN, H), lambda i: (i, 0)),
        out_shape=jax.ShapeDtypeStruct((N, H), jnp.float32),
    )(p, hwp, dinv2, b_prev, W_next)


def _tc_pool_head(p, hwp, dinv2, b_prev, batch2, W1, b1, W2t, b2):
    nsteps = N // BN

    def body(p_ref, hw_ref, di_ref, b_ref, bat_ref, w1_ref, b1_ref,
             w2_ref, b2_ref, o_ref, sums, counts):
        i = pl.program_id(0)

        @pl.when(i == 0)
        def _():
            sums[...] = jnp.zeros_like(sums)
            counts[...] = jnp.zeros_like(counts)

        di = di_ref[...]
        h = jnp.maximum(di * (p_ref[0] + p_ref[1] + hw_ref[...]) + b_ref[...],
                        0.0)
        gids = jax.lax.broadcasted_iota(jnp.int32, (1, G), 1)
        oh = (bat_ref[...] == gids).astype(jnp.float32)
        sums[...] += lax.dot_general(oh, h, (((0,), (0,)), ((), ())),
                                     preferred_element_type=jnp.float32)
        counts[...] += jnp.sum(oh, axis=0)[:, None]

        @pl.when(i == nsteps - 1)
        def _():
            g = sums[...] / jnp.maximum(counts[...], 1.0)
            z = jnp.maximum(
                jnp.dot(g, w1_ref[...],
                        preferred_element_type=jnp.float32) + b1_ref[...], 0.0)
            o_ref[...] = (jnp.sum(z * w2_ref[...], axis=1, keepdims=True)
                          + b2_ref[...])

    return pl.pallas_call(
        body,
        grid=(nsteps,),
        in_specs=[
            pl.BlockSpec((NC, BN, H), lambda i: (0, i, 0)),
            pl.BlockSpec((BN, H), lambda i: (i, 0)),
            pl.BlockSpec((BN, 1), lambda i: (i, 0)),
            pl.BlockSpec((H,), lambda i: (0,)),
            pl.BlockSpec((BN, 1), lambda i: (i, 0)),
            pl.BlockSpec((H, H), lambda i: (0, 0)),
            pl.BlockSpec((H,), lambda i: (0,)),
            pl.BlockSpec((1, H), lambda i: (0, 0)),
            pl.BlockSpec((1,), lambda i: (0,)),
        ],
        out_specs=pl.BlockSpec((G, 1), lambda i: (0, 0)),
        out_shape=jax.ShapeDtypeStruct((G, 1), jnp.float32),
        scratch_shapes=[
            pltpu.VMEM((G, H), jnp.float32),
            pltpu.VMEM((G, 1), jnp.float32),
        ],
    )(p, hwp, dinv2, b_prev, batch2, W1, b1, W2t, b2)


def kernel(x, edge_index, edge_attr, batch, W_node, b_node, conv_W, conv_b,
           We1, be1, We2, be2, W1, b1, W2, b2):
    src = edge_index[0]
    dst = edge_index[1]

    degp = _sc_degree(dst)
    ew = _tc_edge_mlp(edge_attr, We1, be1, We2, be2)
    dinv2 = _tc_dinv(degp)

    hwp = _tc_embed(x, W_node, b_node, conv_W[0], dinv2)
    for l in range(2):
        p = _sc_gather_mul_scatter(hwp, ew, src, dst)
        hwp = _tc_layer_update(p, hwp, dinv2, conv_b[l], conv_W[l + 1])
    p = _sc_gather_mul_scatter(hwp, ew, src, dst)

    return _tc_pool_head(p, hwp, dinv2, conv_b[2], batch[:, None],
                         W1, b1, W2.reshape(1, H), b2)
